# baseline (speedup 1.0000x reference)
import functools
import jax
import jax.numpy as jnp
from jax import lax
from jax.experimental import pallas as pl
from jax.experimental.pallas import tpu as pltpu
from jax.experimental.pallas import tpu_sc as plsc

VOCAB = 100000
EMBED_DIM = 64
BATCH = 16384

_NC = 2
_NS = 16
_NW = _NC * _NS
_B_PER_W = BATCH // _NW
_CHUNK = 128
_N_CHUNKS = _B_PER_W // _CHUNK
_L = 16

_NGRP = VOCAB // 8
_G_PER_W = 392
_GBLK = 28
_A_ITERS = _G_PER_W // _GBLK
_G_LAST = _NGRP - _G_PER_W * (_NW - 1)
_G_TAIL0 = _G_PER_W * (_NW - 1) + (_G_LAST // _GBLK) * _GBLK
_G_TAIL = _NGRP - _G_TAIL0

_mesh = plsc.VectorSubcoreMesh(core_axis_name="c", subcore_axis_name="s")
_params = pltpu.CompilerParams(needs_layout_passes=False)


@functools.partial(
    pl.kernel,
    out_type=jax.ShapeDtypeStruct((VOCAB // 32, 16, 128), jnp.float32),
    mesh=_mesh,
    scratch_types=[
        pltpu.VMEM((2, _GBLK, 8, EMBED_DIM), jnp.float32),
        pltpu.VMEM((2, _GBLK // 4, 16, 128), jnp.float32),
        pltpu.SemaphoreType.DMA,
    ],
    compiler_params=_params,
)
def _depad_table(wp_hbm, out_hbm, a_v, b_v, isem):
    wid = lax.axis_index("s") * _NC + lax.axis_index("c")
    g0 = wid * _G_PER_W
    gend = lax.min(g0 + _G_PER_W, jnp.int32(_G_TAIL0))

    def fire(k, buf, blk, gbase):
        g = gbase + k * _GBLK

        @pl.when(g + blk <= gend)
        def _():
            pltpu.async_copy(
                wp_hbm.at[pl.ds(g, blk)], a_v.at[buf, pl.ds(0, blk)], isem
            )

    def squeeze(buf, blk):
        src = a_v.at[buf]
        dst = b_v.at[buf]

        @plsc.parallel_loop(0, blk * 8, 1, unroll=8)
        def _(t):
            h = lax.bitwise_and(t, 1)
            p = lax.shift_right_logical(t, 1)
            v = lax.shift_right_logical(p, 4)
            s = lax.bitwise_and(p, 15)
            for c in range(EMBED_DIM // _L):
                x = src[lax.shift_right_logical(t, 3),
                        lax.bitwise_and(t, 7),
                        pl.ds(c * _L, _L)]
                dst[v, s, pl.ds(h * EMBED_DIM + c * _L, _L)] = x

    fire(0, 0, _GBLK, g0)
    for k in range(_A_ITERS):
        buf = k % 2
        fire(k + 1, (k + 1) % 2, _GBLK, g0)
        g = g0 + k * _GBLK

        @pl.when(g + _GBLK <= gend)
        def _():
            pltpu.make_async_copy(
                wp_hbm.at[pl.ds(0, _GBLK)], a_v.at[buf, pl.ds(0, _GBLK)],
                isem,
            ).wait()
            squeeze(buf, _GBLK)
            pltpu.sync_copy(
                b_v.at[buf],
                out_hbm.at[pl.ds(lax.shift_right_logical(g, 2), _GBLK // 4)],
            )

    @pl.when(wid == _NW - 1)
    def _():
        pltpu.sync_copy(
            wp_hbm.at[pl.ds(_G_TAIL0, _G_TAIL)],
            a_v.at[0, pl.ds(0, _G_TAIL)],
        )
        squeeze(0, _G_TAIL)
        pltpu.sync_copy(
            b_v.at[0, pl.ds(0, _G_TAIL // 4)],
            out_hbm.at[pl.ds(_G_TAIL0 // 4, _G_TAIL // 4)],
        )


@functools.partial(
    pl.kernel,
    out_type=jax.ShapeDtypeStruct((EMBED_DIM, BATCH), jnp.float32),
    mesh=_mesh,
    scratch_types=[
        pltpu.VMEM((_B_PER_W,), jnp.int32),
        pltpu.VMEM((_B_PER_W,), jnp.int32),
        pltpu.VMEM((_B_PER_W,), jnp.int32),
        pltpu.VMEM((_N_CHUNKS, _CHUNK, 128), jnp.float32),
        pltpu.VMEM((EMBED_DIM, _CHUNK), jnp.float32),
        pltpu.SemaphoreType.DMA,
    ],
    compiler_params=_params,
)
def _gather_rows(idx_hbm, table_hbm, out_hbm, idx_v, v_v, s_v, blocks_v,
                 ot_v, gsem):
    wid = lax.axis_index("s") * _NC + lax.axis_index("c")
    base = wid * _B_PER_W

    pltpu.sync_copy(idx_hbm.at[wid], idx_v)

    for k in range(_B_PER_W // _L):
        ivec = idx_v[pl.ds(k * _L, _L)]
        v_v[pl.ds(k * _L, _L)] = lax.shift_right_logical(ivec, 1)
        s_v[pl.ds(k * _L, _L)] = lax.shift_left(lax.bitwise_and(ivec, 1), 6)

    iota = lax.iota(jnp.int32, _L)
    i0s = [lg * 16 + iota for lg in range(_CHUNK // _L)]

    def fire(c, buf):
        return pltpu.async_copy(
            table_hbm.at[v_v.at[pl.ds(c * _CHUNK, _CHUNK)]],
            blocks_v.at[buf],
            gsem,
        )

    def extract(c, buf):
        src = blocks_v.at[buf]
        for lg in range(_CHUNK // _L):
            i0 = i0s[lg]
            s64 = s_v[pl.ds(c * _CHUNK + lg * 16, _L)]

            @plsc.parallel_loop(0, EMBED_DIM, 1, unroll=8)
            def _(d):
                x = plsc.load_gather(src, [i0, s64 + d])
                ot_v[d, pl.ds(lg * _L, _L)] = x

    gathers = [fire(c, c) for c in range(_N_CHUNKS)]
    for c in range(_N_CHUNKS):
        gathers[c].wait()
        extract(c, c)
        pltpu.sync_copy(
            ot_v, out_hbm.at[:, pl.ds(base + c * _CHUNK, _CHUNK)]
        )


def kernel(inputs, W):
    idx = inputs.reshape(_NW, _B_PER_W)
    wp = W.reshape(_NGRP, 8, EMBED_DIM)
    table = _depad_table(wp).reshape(VOCAB // 2, 2 * EMBED_DIM)
    return _gather_rows(idx, table).T
